# baseline (speedup 1.0000x reference)
import functools
import math

import numpy as np

import jax
import jax.numpy as jnp
from jax import lax
from jax.experimental import pallas as pl
from jax.experimental.pallas import tpu as pltpu
from jax.experimental.pallas import tpu_sc as plsc

N = 10000
E = 320000
ND = 128
G = 512

_COLA = np.array([32 * (w // 16) + w % 16 for w in range(64)])
_COLB = _COLA + 16


def _pack_cols(pa, pb):
    a = lax.bitcast_convert_type(
        pa.astype(jnp.bfloat16).astype(jnp.float32), jnp.int32)
    b = lax.bitcast_convert_type(
        pb.astype(jnp.bfloat16).astype(jnp.float32), jnp.int32)
    return lax.shift_right_logical(a, 16) | (b & jnp.int32(-65536))

_BN = 1000


def _enc_node_body(x_ref, o_ref):
    xb = x_ref[...]
    atom = xb[:, 0:1].astype(jnp.int32)
    cols = lax.broadcasted_iota(jnp.int32, (_BN, ND), 1)
    onehot = (cols == atom).astype(jnp.float32)
    srows = lax.broadcasted_iota(jnp.int32, (10, ND), 0)
    scols = lax.broadcasted_iota(jnp.int32, (10, ND), 1)
    shift = ((scols == srows + 118) & (srows >= 1)).astype(jnp.float32)
    o_ref[...] = onehot + jnp.dot(xb, shift, preferred_element_type=jnp.float32)


def _enc_node(x):
    return pl.pallas_call(
        _enc_node_body,
        grid=(N // _BN,),
        in_specs=[pl.BlockSpec((_BN, 10), lambda i: (i, 0))],
        out_specs=pl.BlockSpec((_BN, ND), lambda i: (i, 0)),
        out_shape=jax.ShapeDtypeStruct((N, ND), jnp.float32),
    )(x)


_BE2 = 1000
_EH = E // 2


def _proj_half(eb, wa_ref, ba_ref, wb_ref, bb_ref):
    bond = eb[:, 0:1].astype(jnp.int32)
    cols = lax.broadcasted_iota(jnp.int32, (_BE2, 40), 1)
    onehot = (cols == bond).astype(jnp.float32)
    srows = lax.broadcasted_iota(jnp.int32, (16, 40), 0)
    scols = lax.broadcasted_iota(jnp.int32, (16, 40), 1)
    shift = ((scols == srows + 21) & (srows >= 1)).astype(jnp.float32)
    ea40 = (onehot + jnp.dot(eb, shift, preferred_element_type=jnp.float32)
            ).astype(jnp.bfloat16)
    pa = jnp.dot(ea40, wa_ref[...].astype(jnp.bfloat16),
                 preferred_element_type=jnp.float32) + ba_ref[...]
    pb = jnp.dot(ea40, wb_ref[...].astype(jnp.bfloat16),
                 preferred_element_type=jnp.float32) + bb_ref[...]
    return _pack_cols(pa, pb)


def _edge_proj_body(ea1_ref, ea2_ref, wa_ref, ba_ref, wb_ref, bb_ref, o_ref):
    o_ref[:, 0:64] = _proj_half(ea1_ref[...], wa_ref, ba_ref, wb_ref, bb_ref)
    o_ref[:, 64:128] = _proj_half(ea2_ref[...], wa_ref, ba_ref, wb_ref, bb_ref)


def _edge_proj(edge_attr, wl, bl):
    wa = wl[:, _COLA]
    wb = wl[:, _COLB]
    ba = bl[_COLA].reshape(1, 64)
    bb = bl[_COLB].reshape(1, 64)
    nblk = _EH // _BE2
    return pl.pallas_call(
        _edge_proj_body,
        grid=(nblk,),
        in_specs=[
            pl.BlockSpec((_BE2, 16), lambda i: (i, 0)),
            pl.BlockSpec((_BE2, 16), lambda i: (i + _EH // _BE2, 0)),
            pl.BlockSpec((40, 64), lambda i: (0, 0)),
            pl.BlockSpec((1, 64), lambda i: (0, 0)),
            pl.BlockSpec((40, 64), lambda i: (0, 0)),
            pl.BlockSpec((1, 64), lambda i: (0, 0)),
        ],
        out_specs=pl.BlockSpec((_BE2, 128), lambda i: (i, 0)),
        out_shape=jax.ShapeDtypeStruct((_EH, 128), jnp.int32),
    )(edge_attr, edge_attr, wa, ba, wb, bb)


_TILES = 32
_NCHUNK = E // 128
_NFULL = _NCHUNK // _TILES
_NEXTRA = _NCHUNK - _NFULL * _TILES
_RPT = 624
_XTR = N - 16 * _RPT

_mesh = plsc.VectorSubcoreMesh(core_axis_name="c", subcore_axis_name="s")


@functools.partial(
    pl.kernel,
    mesh=_mesh,
    out_type=jax.ShapeDtypeStruct((2, N, ND), jnp.float32),
    scratch_types=[
        pltpu.VMEM((128,), jnp.int32),
        pltpu.VMEM((128,), jnp.int32),
        pltpu.VMEM((64, ND), jnp.int32),
        pltpu.VMEM((128, ND), jnp.float32),
        pltpu.VMEM((128,), jnp.int32),
        pltpu.VMEM((128,), jnp.int32),
        pltpu.VMEM((64, ND), jnp.int32),
        pltpu.VMEM((128, ND), jnp.float32),
        pltpu.VMEM_SHARED((N, ND), jnp.float32),
        pltpu.SemaphoreType.DMA,
        pltpu.SemaphoreType.DMA,
        pltpu.SemaphoreType.DMA,
        pltpu.SemaphoreType.DMA,
    ],
)
def _msg_pass(h_hbm, ep_hbm, src_hbm, dst_hbm, out_hbm,
              srcva, dstva, pbufa, gbufa, srcvb, dstvb, pbufb, gbufb,
              aggr, semfa, semga, semfb, semgb):
    c = lax.axis_index("c")
    s = lax.axis_index("s")
    t = c * 16 + s
    bufs = ((srcva, dstva, pbufa, gbufa, semfa, semga),
            (srcvb, dstvb, pbufb, gbufb, semfb, semgb))

    def issue_front(cg, b):
        sv, dv, pb, _, sf, _ = bufs[b]
        base = cg * 64
        pltpu.async_copy(src_hbm.at[pl.ds(base, 64)], sv.at[pl.ds(0, 64)], sf)
        pltpu.async_copy(src_hbm.at[pl.ds(_EH + base, 64)],
                         sv.at[pl.ds(64, 64)], sf)
        pltpu.async_copy(dst_hbm.at[pl.ds(base, 64)], dv.at[pl.ds(0, 64)], sf)
        pltpu.async_copy(dst_hbm.at[pl.ds(_EH + base, 64)],
                         dv.at[pl.ds(64, 64)], sf)
        pltpu.async_copy(ep_hbm.at[pl.ds(base, 64)], pb, sf)

    def wait_front(b):
        sv, dv, pb, _, sf, _ = bufs[b]
        pltpu.make_async_copy(src_hbm.at[pl.ds(0, 64)],
                              sv.at[pl.ds(0, 64)], sf).wait()
        pltpu.make_async_copy(src_hbm.at[pl.ds(0, 64)],
                              sv.at[pl.ds(64, 64)], sf).wait()
        pltpu.make_async_copy(dst_hbm.at[pl.ds(0, 64)],
                              dv.at[pl.ds(0, 64)], sf).wait()
        pltpu.make_async_copy(dst_hbm.at[pl.ds(0, 64)],
                              dv.at[pl.ds(64, 64)], sf).wait()
        pltpu.make_async_copy(ep_hbm.at[pl.ds(0, 64)], pb, sf).wait()

    def issue_gather(b):
        sv, _, _, gb, _, sg = bufs[b]
        pltpu.async_copy(h_hbm.at[sv], gb, sg)

    def wait_gather(b):
        sv, _, _, gb, _, sg = bufs[b]
        pltpu.make_async_copy(h_hbm.at[sv], gb, sg).wait()

    hibits = jnp.int32(-65536)

    def fuse_rows(pb, gb, npairs=64, unroll=4):
        def fuse_row(r, cc):
            for half in range(2):
                rowi = half * 64 + r
                for jj in range(4):
                    ew = pb[r, pl.ds(half * 64 + jj * 16, 16)]
                    elo = lax.bitcast_convert_type(
                        jnp.left_shift(ew, 16), jnp.float32)
                    ehi = lax.bitcast_convert_type(
                        jnp.bitwise_and(ew, hibits), jnp.float32)
                    va = gb[rowi, pl.ds(jj * 32, 16)]
                    vb = gb[rowi, pl.ds(jj * 32 + 16, 16)]
                    gb[rowi, pl.ds(jj * 32, 16)] = jnp.maximum(va + elo, 0.0)
                    gb[rowi, pl.ds(jj * 32 + 16, 16)] = jnp.maximum(
                        vb + ehi, 0.0)
            return cc

        lax.fori_loop(0, npairs, fuse_row, 0, unroll=unroll)

    issue_front(t, 0)
    issue_front(t + 32, 1)

    zv = jnp.zeros((16,), jnp.float32)

    def zrow(r, carry):
        for j in range(8):
            gbufa[r, pl.ds(j * 16, 16)] = zv
        return carry

    lax.fori_loop(0, 128, zrow, 0)
    row0 = s * _RPT
    for k, nr in ((0, 128), (128, 128), (256, 128), (384, 128), (512, 112)):
        pltpu.sync_copy(gbufa.at[pl.ds(0, nr)], aggr.at[pl.ds(row0 + k, nr)])

    @pl.when(s == 15)
    def _():
        pltpu.sync_copy(gbufa.at[pl.ds(0, _XTR)], aggr.at[pl.ds(16 * _RPT, _XTR)])

    plsc.subcore_barrier()

    wait_front(0)
    issue_gather(0)

    def pairbody(i, carry):
        for boff in range(2):
            j = 2 * i + boff
            b = boff
            wait_gather(b)

            @pl.when(j + 1 < _NFULL)
            def _():
                wait_front(1 - b)
                issue_gather(1 - b)

            sv, dv, pb, gb, _, _ = bufs[b]
            fuse_rows(pb, gb)
            pltpu.sync_copy(gb, aggr.at[dv], add=True)

            @pl.when(j + 2 < _NFULL)
            def _():
                issue_front(t + 32 * (j + 2), b)

        return carry

    lax.fori_loop(0, _NFULL // 2, pairbody, 0)

    @pl.when(t < _NEXTRA)
    def _():
        cg = 32 * _NFULL + t
        issue_front(cg, 0)
        wait_front(0)
        pltpu.sync_copy(h_hbm.at[srcva], gbufa)
        fuse_rows(pbufa, gbufa)
        pltpu.sync_copy(gbufa, aggr.at[dstva], add=True)

    plsc.subcore_barrier()
    pltpu.sync_copy(aggr.at[pl.ds(row0, _RPT)], out_hbm.at[c, pl.ds(row0, _RPT)])

    @pl.when(s == 15)
    def _():
        pltpu.sync_copy(aggr.at[pl.ds(16 * _RPT, _XTR)],
                        out_hbm.at[c, pl.ds(16 * _RPT, _XTR)])


def _update_body(h_ref, a_ref, w_ref, b_ref, o_ref):
    tv = h_ref[...] + a_ref[0] + a_ref[1]
    t1 = jnp.dot(tv, w_ref[...], preferred_element_type=jnp.float32) + b_ref[...]
    o_ref[...] = jnp.where(t1 >= 0, t1, 0.01 * t1)


def _update(h, aggr, wn, bn):
    return pl.pallas_call(
        _update_body,
        grid=(N // _BN,),
        in_specs=[
            pl.BlockSpec((_BN, ND), lambda i: (i, 0)),
            pl.BlockSpec((2, _BN, ND), lambda i: (0, i, 0)),
            pl.BlockSpec((ND, ND), lambda i: (0, 0)),
            pl.BlockSpec((1, ND), lambda i: (0, 0)),
        ],
        out_specs=pl.BlockSpec((_BN, ND), lambda i: (i, 0)),
        out_shape=jax.ShapeDtypeStruct((N, ND), jnp.float32),
    )(h, aggr, wn, bn.reshape(1, ND))


def _update_pool_body(b_ref, h_ref, a_ref, w_ref, bias_ref, o_ref):
    i = pl.program_id(0)
    tv = h_ref[...] + a_ref[0] + a_ref[1]
    t1 = jnp.dot(tv, w_ref[...], preferred_element_type=jnp.float32) + bias_ref[...]
    hn = jnp.where(t1 >= 0, t1, 0.01 * t1)
    bb = b_ref[0]
    g_iota = lax.broadcasted_iota(jnp.int32, (G, _BN), 0)
    sel = (g_iota == bb).astype(jnp.float32)
    contrib = jnp.dot(sel, hn, preferred_element_type=jnp.float32)

    @pl.when(i == 0)
    def _():
        o_ref[...] = jnp.zeros_like(o_ref)

    o_ref[...] += contrib


def _update_pool(batch3, h, aggr, wn, bn):
    return pl.pallas_call(
        _update_pool_body,
        grid=(N // _BN,),
        in_specs=[
            pl.BlockSpec((1, 1, _BN), lambda i: (i, 0, 0)),
            pl.BlockSpec((_BN, ND), lambda i: (i, 0)),
            pl.BlockSpec((2, _BN, ND), lambda i: (0, i, 0)),
            pl.BlockSpec((ND, ND), lambda i: (0, 0)),
            pl.BlockSpec((1, ND), lambda i: (0, 0)),
        ],
        out_specs=pl.BlockSpec((G, ND), lambda i: (0, 0)),
        out_shape=jax.ShapeDtypeStruct((G, ND), jnp.float32),
    )(batch3, h, aggr, wn, bn)


_INV = 1.0 / math.sqrt(1.0 + 1e-5)


def _head_body(p_ref, g1_ref, bt1_ref, w1_ref, b1_ref, g2_ref, bt2_ref,
               w2_ref, b2_ref, o_ref):
    z = p_ref[...] * (_INV * g1_ref[...]) + bt1_ref[...]
    z = jnp.dot(z, w1_ref[...], preferred_element_type=jnp.float32) + b1_ref[...]
    z = jnp.maximum(z, 0.0)
    z = z * (_INV * g2_ref[...]) + bt2_ref[...]
    o_ref[...] = jnp.dot(z, w2_ref[...], preferred_element_type=jnp.float32) + b2_ref[...]


def _head(pooled, g1, bt1, w1, b1, g2, bt2, w2, b2):
    def full(shape):
        return pl.BlockSpec(shape, lambda: tuple(0 for _ in shape))

    return pl.pallas_call(
        _head_body,
        in_specs=[full((G, ND)), full((1, ND)), full((1, ND)), full((ND, 64)),
                  full((1, 64)), full((1, 64)), full((1, 64)), full((64, 2)),
                  full((1, 2))],
        out_specs=full((G, 2)),
        out_shape=jax.ShapeDtypeStruct((G, 2), jnp.float32),
    )(pooled, g1, bt1, w1, b1, g2, bt2, w2, b2)


def kernel(x, edge_index, edge_attr, batch,
           We0, be0, Wn0, bn0, We1, be1, Wn1, bn1, We2, be2, Wn2, bn2,
           g1, bt1, Wh1, bh1, g2, bt2, Wh2, bh2):
    src = edge_index[0].astype(jnp.int32)
    dst = edge_index[1].astype(jnp.int32)

    h = _enc_node(x)
    batch3 = batch.astype(jnp.int32).reshape(N // _BN, 1, _BN)

    wp = [jnp.pad(w, ((0, 3), (0, 0))) for w in (We0, We1, We2)]

    ep0 = _edge_proj(edge_attr, wp[0], be0)
    aggr = _msg_pass(h, ep0, src, dst)
    ep1 = _edge_proj(edge_attr, wp[1], be1)
    h = _update(h, aggr, Wn0, bn0)
    aggr = _msg_pass(h, ep1, src, dst)
    ep2 = _edge_proj(edge_attr, wp[2], be2)
    h = _update(h, aggr, Wn1, bn1)
    aggr = _msg_pass(h, ep2, src, dst)
    pooled = _update_pool(batch3, h, aggr, Wn2, bn2.reshape(1, ND))

    return _head(pooled, g1.reshape(1, ND), bt1.reshape(1, ND), Wh1,
                 bh1.reshape(1, 64), g2.reshape(1, 64), bt2.reshape(1, 64),
                 Wh2, bh2.reshape(1, 2))


# speedup vs baseline: 1.5566x; 1.5566x over previous
import functools
import math

import jax
import jax.numpy as jnp
from jax import lax
from jax.experimental import pallas as pl
from jax.experimental.pallas import tpu as pltpu
from jax.experimental.pallas import tpu_sc as plsc

N = 10000
E = 320000
ND = 128
G = 512

_BN = 1000


def _enc_node_body(x_ref, o_ref):
    xb = x_ref[...]
    atom = xb[:, 0:1].astype(jnp.int32)
    cols = lax.broadcasted_iota(jnp.int32, (_BN, ND), 1)
    onehot = (cols == atom).astype(jnp.float32)
    srows = lax.broadcasted_iota(jnp.int32, (10, ND), 0)
    scols = lax.broadcasted_iota(jnp.int32, (10, ND), 1)
    shift = ((scols == srows + 118) & (srows >= 1)).astype(jnp.float32)
    o_ref[...] = onehot + jnp.dot(xb, shift, preferred_element_type=jnp.float32)


def _enc_node(x):
    return pl.pallas_call(
        _enc_node_body,
        grid=(N // _BN,),
        in_specs=[pl.BlockSpec((_BN, 10), lambda i: (i, 0))],
        out_specs=pl.BlockSpec((_BN, ND), lambda i: (i, 0)),
        out_shape=jax.ShapeDtypeStruct((N, ND), jnp.float32),
    )(x)


_BE = 2000


def _edge_proj_body(ea_ref, w_ref, b_ref, o_ref):
    eb = ea_ref[...]
    bond = eb[:, 0:1].astype(jnp.int32)
    cols = lax.broadcasted_iota(jnp.int32, (_BE, 40), 1)
    onehot = (cols == bond).astype(jnp.float32)
    srows = lax.broadcasted_iota(jnp.int32, (16, 40), 0)
    scols = lax.broadcasted_iota(jnp.int32, (16, 40), 1)
    shift = ((scols == srows + 21) & (srows >= 1)).astype(jnp.float32)
    ea40 = onehot + jnp.dot(eb, shift, preferred_element_type=jnp.float32)
    o_ref[...] = (jnp.dot(ea40.astype(jnp.bfloat16),
                          w_ref[...].astype(jnp.bfloat16),
                          preferred_element_type=jnp.float32) + b_ref[...])


def _edge_proj(edge_attr, wl, bl):
    return pl.pallas_call(
        _edge_proj_body,
        grid=(E // _BE,),
        in_specs=[
            pl.BlockSpec((_BE, 16), lambda i: (i, 0)),
            pl.BlockSpec((40, ND), lambda i: (0, 0)),
            pl.BlockSpec((1, ND), lambda i: (0, 0)),
        ],
        out_specs=pl.BlockSpec((_BE, ND), lambda i: (i, 0)),
        out_shape=jax.ShapeDtypeStruct((E, ND), jnp.float32),
    )(edge_attr, wl, bl)


_TILES = 32
_EPT = E // _TILES
_NFULL = _EPT // 128
_REM = _EPT - _NFULL * 128
_RPT = 624
_XTR = N - 16 * _RPT

_mesh = plsc.VectorSubcoreMesh(core_axis_name="c", subcore_axis_name="s")


@functools.partial(
    pl.kernel,
    mesh=_mesh,
    out_type=jax.ShapeDtypeStruct((2, N, ND), jnp.float32),
    scratch_types=[
        pltpu.VMEM((128,), jnp.int32),
        pltpu.VMEM((128,), jnp.int32),
        pltpu.VMEM((128, ND), jnp.float32),
        pltpu.VMEM((128,), jnp.int32),
        pltpu.VMEM((128,), jnp.int32),
        pltpu.VMEM((128, ND), jnp.float32),
        pltpu.VMEM((128,), jnp.int32),
        pltpu.VMEM((128,), jnp.int32),
        pltpu.VMEM((128, ND), jnp.float32),
        pltpu.VMEM((_REM,), jnp.int32),
        pltpu.VMEM((_REM,), jnp.int32),
        pltpu.VMEM_SHARED((N, ND), jnp.float32),
        pltpu.SemaphoreType.DMA,
        pltpu.SemaphoreType.DMA,
        pltpu.SemaphoreType.DMA,
        pltpu.SemaphoreType.DMA,
        pltpu.SemaphoreType.DMA,
        pltpu.SemaphoreType.DMA,
        pltpu.SemaphoreType.DMA,
        pltpu.SemaphoreType.DMA,
        pltpu.SemaphoreType.DMA,
    ],
)
def _msg_pass(h_hbm, ep_hbm, src_hbm, dst_hbm, out_hbm,
              srcva, dstva, mbufa, srcvb, dstvb, mbufb, srcvc, dstvc, mbufc,
              srcr, dstr, aggr,
              semfa, semga, semsa, semfb, semgb, semsb, semfc, semgc, semsc):
    c = lax.axis_index("c")
    s = lax.axis_index("s")
    t = c * 16 + s
    base0 = t * _EPT
    bufs = ((srcva, dstva, mbufa, semfa, semga, semsa),
            (srcvb, dstvb, mbufb, semfb, semgb, semsb),
            (srcvc, dstvc, mbufc, semfc, semgc, semsc))

    def issue_front(j, b):
        sv, dv, mb, sf, _, _ = bufs[b]
        base = base0 + j * 128
        pltpu.async_copy(src_hbm.at[pl.ds(base, 128)], sv, sf)
        pltpu.async_copy(dst_hbm.at[pl.ds(base, 128)], dv, sf)
        pltpu.async_copy(ep_hbm.at[pl.ds(base, 128)], mb, sf)

    def wait_front(b):
        sv, dv, mb, sf, _, _ = bufs[b]
        pltpu.make_async_copy(src_hbm.at[pl.ds(0, 128)], sv, sf).wait()
        pltpu.make_async_copy(dst_hbm.at[pl.ds(0, 128)], dv, sf).wait()
        pltpu.make_async_copy(ep_hbm.at[pl.ds(0, 128)], mb, sf).wait()

    def issue_gather(b):
        sv, _, mb, _, sg, _ = bufs[b]
        pltpu.async_copy(h_hbm.at[sv], mb, sg, add=True)

    def wait_gather(b):
        sv, _, mb, _, sg, _ = bufs[b]
        pltpu.make_async_copy(h_hbm.at[sv], mb, sg).wait()

    def relu_inplace(mb, nrows, unroll=8):
        @functools.partial(plsc.parallel_loop, 0, nrows, unroll=unroll)
        def _relu_row(r):
            for jj in range(8):
                v = mb[r, pl.ds(jj * 16, 16)]
                mb[r, pl.ds(jj * 16, 16)] = jnp.maximum(v, 0.0)

    def issue_scatter(b):
        _, dv, mb, _, _, ss = bufs[b]
        pltpu.async_copy(mb, aggr.at[dv], ss, add=True)

    def wait_scatter(b):
        _, dv, mb, _, _, ss = bufs[b]
        pltpu.make_async_copy(mb, aggr.at[dv], ss).wait()

    zv = jnp.zeros((16,), jnp.float32)

    @functools.partial(plsc.parallel_loop, 0, 128, unroll=8)
    def _zrow(r):
        for j in range(8):
            mbufc[r, pl.ds(j * 16, 16)] = zv

    row0 = s * _RPT
    for k, nr in ((0, 128), (128, 128), (256, 128), (384, 128), (512, 112)):
        pltpu.sync_copy(mbufc.at[pl.ds(0, nr)], aggr.at[pl.ds(row0 + k, nr)])

    @pl.when(s == 15)
    def _():
        pltpu.sync_copy(mbufc.at[pl.ds(0, _XTR)], aggr.at[pl.ds(16 * _RPT, _XTR)])

    plsc.subcore_barrier()

    issue_front(0, 0)
    issue_front(1, 1)
    wait_front(0)
    issue_gather(0)

    def tribody(i, carry):
        for boff in range(3):
            j = 3 * i + boff
            b = boff
            wait_gather(b)

            @pl.when(j + 1 < _NFULL)
            def _():
                wait_front((boff + 1) % 3)
                issue_gather((boff + 1) % 3)

            relu_inplace(bufs[b][2], 128)
            issue_scatter(b)

            @pl.when(jnp.logical_and(j >= 1, j + 2 < _NFULL))
            def _():
                wait_scatter((boff + 2) % 3)

            @pl.when(j + 2 < _NFULL)
            def _():
                issue_front(j + 2, (boff + 2) % 3)

        return carry

    lax.fori_loop(0, _NFULL // 3, tribody, 0)
    wait_scatter(0)
    wait_scatter(1)
    wait_scatter(2)

    rbase = base0 + _NFULL * 128
    pltpu.sync_copy(src_hbm.at[pl.ds(rbase, _REM)], srcr)
    pltpu.sync_copy(dst_hbm.at[pl.ds(rbase, _REM)], dstr)
    pltpu.sync_copy(ep_hbm.at[pl.ds(rbase, _REM)], mbufa.at[pl.ds(0, _REM)])
    pltpu.sync_copy(h_hbm.at[srcr], mbufa.at[pl.ds(0, _REM)], add=True)
    relu_inplace(mbufa, _REM)
    pltpu.sync_copy(mbufa.at[pl.ds(0, _REM)], aggr.at[dstr], add=True)

    plsc.subcore_barrier()
    pltpu.sync_copy(aggr.at[pl.ds(row0, _RPT)], out_hbm.at[c, pl.ds(row0, _RPT)])

    @pl.when(s == 15)
    def _():
        pltpu.sync_copy(aggr.at[pl.ds(16 * _RPT, _XTR)],
                        out_hbm.at[c, pl.ds(16 * _RPT, _XTR)])


def _update_body(h_ref, a_ref, w_ref, b_ref, o_ref):
    tv = h_ref[...] + a_ref[0] + a_ref[1]
    t1 = jnp.dot(tv, w_ref[...], preferred_element_type=jnp.float32) + b_ref[...]
    o_ref[...] = jnp.where(t1 >= 0, t1, 0.01 * t1)


def _update(h, aggr, wn, bn):
    return pl.pallas_call(
        _update_body,
        grid=(N // _BN,),
        in_specs=[
            pl.BlockSpec((_BN, ND), lambda i: (i, 0)),
            pl.BlockSpec((2, _BN, ND), lambda i: (0, i, 0)),
            pl.BlockSpec((ND, ND), lambda i: (0, 0)),
            pl.BlockSpec((1, ND), lambda i: (0, 0)),
        ],
        out_specs=pl.BlockSpec((_BN, ND), lambda i: (i, 0)),
        out_shape=jax.ShapeDtypeStruct((N, ND), jnp.float32),
    )(h, aggr, wn, bn.reshape(1, ND))


def _update_pool_body(b_ref, h_ref, a_ref, w_ref, bias_ref, o_ref):
    i = pl.program_id(0)
    tv = h_ref[...] + a_ref[0] + a_ref[1]
    t1 = jnp.dot(tv, w_ref[...], preferred_element_type=jnp.float32) + bias_ref[...]
    hn = jnp.where(t1 >= 0, t1, 0.01 * t1)
    bb = b_ref[0]
    g_iota = lax.broadcasted_iota(jnp.int32, (G, _BN), 0)
    sel = (g_iota == bb).astype(jnp.float32)
    contrib = jnp.dot(sel, hn, preferred_element_type=jnp.float32)

    @pl.when(i == 0)
    def _():
        o_ref[...] = jnp.zeros_like(o_ref)

    o_ref[...] += contrib


def _update_pool(batch3, h, aggr, wn, bn):
    return pl.pallas_call(
        _update_pool_body,
        grid=(N // _BN,),
        in_specs=[
            pl.BlockSpec((1, 1, _BN), lambda i: (i, 0, 0)),
            pl.BlockSpec((_BN, ND), lambda i: (i, 0)),
            pl.BlockSpec((2, _BN, ND), lambda i: (0, i, 0)),
            pl.BlockSpec((ND, ND), lambda i: (0, 0)),
            pl.BlockSpec((1, ND), lambda i: (0, 0)),
        ],
        out_specs=pl.BlockSpec((G, ND), lambda i: (0, 0)),
        out_shape=jax.ShapeDtypeStruct((G, ND), jnp.float32),
    )(batch3, h, aggr, wn, bn)


_INV = 1.0 / math.sqrt(1.0 + 1e-5)


def _head_body(p_ref, g1_ref, bt1_ref, w1_ref, b1_ref, g2_ref, bt2_ref,
               w2_ref, b2_ref, o_ref):
    z = p_ref[...] * (_INV * g1_ref[...]) + bt1_ref[...]
    z = jnp.dot(z, w1_ref[...], preferred_element_type=jnp.float32) + b1_ref[...]
    z = jnp.maximum(z, 0.0)
    z = z * (_INV * g2_ref[...]) + bt2_ref[...]
    o_ref[...] = jnp.dot(z, w2_ref[...], preferred_element_type=jnp.float32) + b2_ref[...]


def _head(pooled, g1, bt1, w1, b1, g2, bt2, w2, b2):
    def full(shape):
        return pl.BlockSpec(shape, lambda: tuple(0 for _ in shape))

    return pl.pallas_call(
        _head_body,
        in_specs=[full((G, ND)), full((1, ND)), full((1, ND)), full((ND, 64)),
                  full((1, 64)), full((1, 64)), full((1, 64)), full((64, 2)),
                  full((1, 2))],
        out_specs=full((G, 2)),
        out_shape=jax.ShapeDtypeStruct((G, 2), jnp.float32),
    )(pooled, g1, bt1, w1, b1, g2, bt2, w2, b2)


def kernel(x, edge_index, edge_attr, batch,
           We0, be0, Wn0, bn0, We1, be1, Wn1, bn1, We2, be2, Wn2, bn2,
           g1, bt1, Wh1, bh1, g2, bt2, Wh2, bh2):
    src = edge_index[0].astype(jnp.int32)
    dst = edge_index[1].astype(jnp.int32)

    h = _enc_node(x)
    batch3 = batch.astype(jnp.int32).reshape(N // _BN, 1, _BN)

    wp = [jnp.pad(w, ((0, 3), (0, 0))) for w in (We0, We1, We2)]
    bp = [b.reshape(1, ND) for b in (be0, be1, be2)]

    ep0 = _edge_proj(edge_attr, wp[0], bp[0])
    aggr = _msg_pass(h, ep0, src, dst)
    ep1 = _edge_proj(edge_attr, wp[1], bp[1])
    h = _update(h, aggr, Wn0, bn0)
    aggr = _msg_pass(h, ep1, src, dst)
    ep2 = _edge_proj(edge_attr, wp[2], bp[2])
    h = _update(h, aggr, Wn1, bn1)
    aggr = _msg_pass(h, ep2, src, dst)
    pooled = _update_pool(batch3, h, aggr, Wn2, bn2.reshape(1, ND))

    return _head(pooled, g1.reshape(1, ND), bt1.reshape(1, ND), Wh1,
                 bh1.reshape(1, 64), g2.reshape(1, 64), bt2.reshape(1, 64),
                 Wh2, bh2.reshape(1, 2))
